# baseline (speedup 1.0000x reference)
import functools

import jax
import jax.numpy as jnp
from jax import lax
from jax.experimental import pallas as pl
from jax.experimental.pallas import tpu as pltpu
from jax.experimental.pallas import tpu_sc as plsc

N = 10000
NPAD = 10240
E = 160000
EPAD = 163840
IN_DIM = 256
H1 = 256
H2 = 128

NC = 2
NS = 16
L = 16

ECHUNK = 128
CA = 64
DEG_NCHUNK = EPAD // (NC * NS * ECHUNK)
AGG_NCHUNK = EPAD // (NS * CA)
SLAB = NPAD // NS
WB_CHUNK = 128

_sc_mesh = plsc.VectorSubcoreMesh(core_axis_name="c", subcore_axis_name="s")


def _deg_body(dst_hbm, ew_hbm, z1_hbm, deg0_out, deg1_out,
              dst_v, ew_v, zb, shared_deg):
    c = lax.axis_index("c")
    s = lax.axis_index("s")
    wid = c * NS + s

    pltpu.sync_copy(dst_hbm.at[wid], dst_v)
    pltpu.sync_copy(ew_hbm.at[wid], ew_v)

    pltpu.sync_copy(z1_hbm, shared_deg.at[pl.ds(s * SLAB, SLAB)])
    plsc.subcore_barrier()

    def chunk_body(i, _):
        pltpu.sync_copy(ew_v.at[i], shared_deg.at[dst_v.at[i]], add=True)
        return 0
    lax.fori_loop(0, DEG_NCHUNK, chunk_body, 0)
    plsc.subcore_barrier()

    pltpu.sync_copy(shared_deg.at[pl.ds(s * SLAB, SLAB)], zb)

    @pl.when(c == 0)
    def _():
        pltpu.sync_copy(zb, deg0_out.at[pl.ds(s * SLAB, SLAB)])

    @pl.when(c == 1)
    def _():
        pltpu.sync_copy(zb, deg1_out.at[pl.ds(s * SLAB, SLAB)])


@functools.partial(
    pl.kernel,
    out_type=(jax.ShapeDtypeStruct((NPAD,), jnp.float32),
              jax.ShapeDtypeStruct((NPAD,), jnp.float32)),
    mesh=_sc_mesh,
    scratch_types=[
        pltpu.VMEM((DEG_NCHUNK, ECHUNK), jnp.int32),
        pltpu.VMEM((DEG_NCHUNK, ECHUNK), jnp.float32),
        pltpu.VMEM((SLAB,), jnp.float32),
        pltpu.VMEM_SHARED((NPAD,), jnp.float32),
    ],
)
def _deg_call(dst_hbm, ew_hbm, z1_hbm, deg0_out, deg1_out,
              dst_v, ew_v, zb, shared_deg):
    _deg_body(dst_hbm, ew_hbm, z1_hbm, deg0_out, deg1_out,
              dst_v, ew_v, zb, shared_deg)


def _agg_body(ga_hbm, gb_hbm, src_hbm, dst_hbm, ewb_hbm, z2_hbm,
              outa_hbm, outb_hbm,
              idx4, wrow2, rows2,
              semg_a, semg_b, sems_a, sems_b, semw_a, semw_b,
              acc):
    c = lax.axis_index("c")
    s = lax.axis_index("s")

    pltpu.sync_copy(z2_hbm, acc.at[pl.ds(s * SLAB, SLAB)])
    plsc.subcore_barrier()

    def scale_rows(p):
        @plsc.parallel_loop(0, CA, step=1, unroll=4)
        def _(r):
            row = p * CA + r
            w = wrow2[row, pl.ds(0, L)]
            for f in range(H2 // L):
                rows2[row, pl.ds(f * L, L)] = rows2[row, pl.ds(f * L, L)] * w

    def slot(p):
        return (idx4.at[2 * p], idx4.at[2 * p + 1],
                wrow2.at[pl.ds(p * CA, CA)],
                rows2.at[pl.ds(p * CA, CA)],
                (semg_a, semg_b)[p], (sems_a, sems_b)[p],
                (semw_a, semw_b)[p])

    def edge_loop(g_ref):
        def issue(i, p):
            src_c, dst_c, wrow, rows, semg, sems, semw = slot(p)
            pltpu.async_copy(ewb_hbm.at[s, i], wrow, semw)
            pltpu.sync_copy(dst_hbm.at[s, i], dst_c)
            pltpu.sync_copy(src_hbm.at[s, i], src_c)
            pltpu.async_copy(g_ref.at[src_c], rows, semg)

        def process(i, p):
            src_c, dst_c, wrow, rows, semg, sems, semw = slot(p)
            pltpu.make_async_copy(g_ref.at[src_c], rows, semg).wait()
            pltpu.make_async_copy(ewb_hbm.at[s, i], wrow, semw).wait()
            scale_rows(p)
            pltpu.async_copy(rows, acc.at[dst_c], sems, add=True)

        def wait_scatter(p):
            src_c, dst_c, wrow, rows, semg, sems, semw = slot(p)
            pltpu.make_async_copy(rows, acc.at[dst_c], sems).wait()

        issue(0, 0)

        def body(j, _):
            i0 = 2 * j

            @pl.when(j > 0)
            def _():
                wait_scatter(1)
            issue(i0 + 1, 1)
            process(i0, 0)
            process(i0 + 1, 1)
            wait_scatter(0)
            issue((i0 + 2) % AGG_NCHUNK, 0)
            return 0
        lax.fori_loop(0, AGG_NCHUNK // 2, body, 0)

        wait_scatter(1)
        src_c, dst_c, wrow, rows, semg, sems, semw = slot(0)
        pltpu.make_async_copy(g_ref.at[src_c], rows, semg).wait()
        pltpu.make_async_copy(ewb_hbm.at[s, 0], wrow, semw).wait()

    @pl.when(c == 0)
    def _():
        edge_loop(ga_hbm)

    @pl.when(c == 1)
    def _():
        edge_loop(gb_hbm)

    plsc.subcore_barrier()

    def wb(out_ref):
        for i in range(SLAB // WB_CHUNK):
            base = s * SLAB + i * WB_CHUNK
            pltpu.sync_copy(acc.at[pl.ds(base, WB_CHUNK)], rows2)
            pltpu.sync_copy(rows2, out_ref.at[pl.ds(base, WB_CHUNK)])

    @pl.when(c == 0)
    def _():
        wb(outa_hbm)

    @pl.when(c == 1)
    def _():
        wb(outb_hbm)


@functools.partial(
    pl.kernel,
    out_type=(jax.ShapeDtypeStruct((NPAD, H2), jnp.float32),
              jax.ShapeDtypeStruct((NPAD, H2), jnp.float32)),
    mesh=_sc_mesh,
    scratch_types=[
        pltpu.VMEM((4, CA), jnp.int32),
        pltpu.VMEM((2 * CA, L), jnp.float32),
        pltpu.VMEM((2 * CA, H2), jnp.float32),
        pltpu.SemaphoreType.DMA,
        pltpu.SemaphoreType.DMA,
        pltpu.SemaphoreType.DMA,
        pltpu.SemaphoreType.DMA,
        pltpu.SemaphoreType.DMA,
        pltpu.SemaphoreType.DMA,
        pltpu.VMEM_SHARED((NPAD, H2), jnp.float32),
    ],
)
def _agg_call(ga_hbm, gb_hbm, src_hbm, dst_hbm, ewb_hbm, z2_hbm,
              outa_hbm, outb_hbm,
              idx4, wrow2, rows2,
              semg_a, semg_b, sems_a, sems_b, semw_a, semw_b,
              acc):
    _agg_body(ga_hbm, gb_hbm, src_hbm, dst_hbm, ewb_hbm, z2_hbm,
              outa_hbm, outb_hbm,
              idx4, wrow2, rows2,
              semg_a, semg_b, sems_a, sems_b, semw_a, semw_b,
              acc)


BR = 1000


def _dinv_body(degp_ref, dinv_ref):
    d = degp_ref[0, :] + degp_ref[1, :] + 1.0
    dinv_ref[...] = lax.rsqrt(d)[:, None]


def _dinv_call(deg_p):
    return pl.pallas_call(
        _dinv_body,
        out_shape=jax.ShapeDtypeStruct((NPAD, 1), jnp.float32),
    )(deg_p)


def _t1_body(x_ref, w_ref, dinv_ref, glo_ref, ghi_ref):
    h = jnp.dot(x_ref[...], w_ref[...], preferred_element_type=jnp.float32)
    g = h * dinv_ref[...]
    glo_ref[...] = g[:, :H2]
    ghi_ref[...] = g[:, H2:]


def _t1_call(x, W0, dinv):
    return pl.pallas_call(
        _t1_body,
        grid=(N // BR,),
        in_specs=[
            pl.BlockSpec((BR, IN_DIM), lambda i: (i, 0)),
            pl.BlockSpec((IN_DIM, H1), lambda i: (0, 0)),
            pl.BlockSpec((BR, 1), lambda i: (i, 0)),
        ],
        out_specs=(
            pl.BlockSpec((BR, H2), lambda i: (i, 0)),
            pl.BlockSpec((BR, H2), lambda i: (i, 0)),
        ),
        out_shape=(jax.ShapeDtypeStruct((N, H2), jnp.float32),
                   jax.ShapeDtypeStruct((N, H2), jnp.float32)),
    )(x, W0, dinv)


def _t2_body(alo_ref, ahi_ref, glo_ref, ghi_ref, dinv_ref, b0_ref, wc_ref,
             g1_ref, g2_ref):
    dinv = dinv_ref[...]
    hlo = (alo_ref[...] + glo_ref[...]) * dinv
    hhi = (ahi_ref[...] + ghi_ref[...]) * dinv
    h = jnp.concatenate([hlo, hhi], axis=1) + b0_ref[...]
    h = jnp.maximum(h, 0.0)
    m = jnp.dot(h, wc_ref[...], preferred_element_type=jnp.float32)
    g1_ref[...] = m[:, :H2] * dinv
    g2_ref[...] = m[:, H2:] * dinv


def _t2_call(alo, ahi, glo, ghi, dinv, b0, Wc):
    return pl.pallas_call(
        _t2_body,
        grid=(N // BR,),
        in_specs=[
            pl.BlockSpec((BR, H2), lambda i: (i, 0)),
            pl.BlockSpec((BR, H2), lambda i: (i, 0)),
            pl.BlockSpec((BR, H2), lambda i: (i, 0)),
            pl.BlockSpec((BR, H2), lambda i: (i, 0)),
            pl.BlockSpec((BR, 1), lambda i: (i, 0)),
            pl.BlockSpec((1, H1), lambda i: (0, 0)),
            pl.BlockSpec((H1, 2 * H2), lambda i: (0, 0)),
        ],
        out_specs=(
            pl.BlockSpec((BR, H2), lambda i: (i, 0)),
            pl.BlockSpec((BR, H2), lambda i: (i, 0)),
        ),
        out_shape=(jax.ShapeDtypeStruct((N, H2), jnp.float32),
                   jax.ShapeDtypeStruct((N, H2), jnp.float32)),
    )(alo, ahi, glo, ghi, dinv, b0, Wc)


def _t3_body(a1_ref, g1_ref, a2_ref, g2_ref, dinv_ref, b1_ref, b2_ref,
             noise_ref, z_ref, zb_ref):
    dinv = dinv_ref[...]
    mean = (a1_ref[...] + g1_ref[...]) * dinv + b1_ref[...]
    log_std = (a2_ref[...] + g2_ref[...]) * dinv + b2_ref[...]
    z = mean + noise_ref[...] * jnp.exp(log_std)
    z_ref[...] = z
    zb_ref[...] = z.astype(jnp.bfloat16)


def _t3_call(a1, g1, a2, g2, dinv, b1, b2, noise):
    return pl.pallas_call(
        _t3_body,
        grid=(N // BR,),
        in_specs=[
            pl.BlockSpec((BR, H2), lambda i: (i, 0)),
            pl.BlockSpec((BR, H2), lambda i: (i, 0)),
            pl.BlockSpec((BR, H2), lambda i: (i, 0)),
            pl.BlockSpec((BR, H2), lambda i: (i, 0)),
            pl.BlockSpec((BR, 1), lambda i: (i, 0)),
            pl.BlockSpec((1, H2), lambda i: (0, 0)),
            pl.BlockSpec((1, H2), lambda i: (0, 0)),
            pl.BlockSpec((BR, H2), lambda i: (i, 0)),
        ],
        out_specs=(pl.BlockSpec((BR, H2), lambda i: (i, 0)),
                   pl.BlockSpec((BR, H2), lambda i: (i, 0))),
        out_shape=(jax.ShapeDtypeStruct((N, H2), jnp.float32),
                   jax.ShapeDtypeStruct((N, H2), jnp.bfloat16)),
    )(a1, g1, a2, g2, dinv, b1, b2, noise)


DEC_BM = 200


def _decoder_body(zi_ref, zj_ref, out_ref):
    acc = jax.lax.dot_general(
        zi_ref[...], zj_ref[...],
        (((1,), (1,)), ((), ())),
        preferred_element_type=jnp.float32,
    )
    out_ref[...] = jax.nn.sigmoid(acc)


def _decoder(zb):
    n = zb.shape[0]
    return pl.pallas_call(
        _decoder_body,
        grid=(n // DEC_BM,),
        in_specs=[
            pl.BlockSpec((DEC_BM, H2), lambda i: (i, 0)),
            pl.BlockSpec((n, H2), lambda i: (0, 0)),
        ],
        out_specs=pl.BlockSpec((DEC_BM, n), lambda i: (i, 0)),
        out_shape=jax.ShapeDtypeStruct((n, n), jnp.float32),
    )(zb, zb)


def kernel(x, edge_index, edge_weight, W0, b0, W1, b1, W2, b2):
    src = edge_index[0]
    dst = edge_index[1]
    n = x.shape[0]

    pad = EPAD - E
    pad_idx = (jnp.arange(pad, dtype=jnp.int32) * 37) % N
    src_p = jnp.concatenate([src, pad_idx])
    dst_p = jnp.concatenate([dst, pad_idx])
    ew_p = jnp.concatenate([edge_weight, jnp.zeros((pad,), jnp.float32)])

    dst_deg = dst_p.reshape(NC * NS, DEG_NCHUNK, ECHUNK)
    ew_deg = ew_p.reshape(NC * NS, DEG_NCHUNK, ECHUNK)
    src_agg = src_p.reshape(NS, AGG_NCHUNK, CA)
    dst_agg = dst_p.reshape(NS, AGG_NCHUNK, CA)
    ew_bc = jnp.broadcast_to(
        ew_p[:, None], (EPAD, L)).reshape(NS, AGG_NCHUNK, CA, L)

    z1 = jnp.zeros((SLAB,), jnp.float32)
    z2 = jnp.zeros((SLAB, H2), jnp.float32)

    deg0, deg1 = _deg_call(dst_deg, ew_deg, z1)
    dinv_pad = _dinv_call(jnp.stack([deg0, deg1]))
    dinv = dinv_pad[:N]

    g0_lo, g0_hi = _t1_call(x, W0, dinv)
    a0_lo, a0_hi = _agg_call(g0_lo, g0_hi, src_agg, dst_agg, ew_bc, z2)
    a0_lo, a0_hi = a0_lo[:N], a0_hi[:N]

    Wc = jnp.concatenate([W1, W2], axis=1)
    g1, g2 = _t2_call(a0_lo, a0_hi, g0_lo, g0_hi, dinv,
                      b0.reshape(1, H1), Wc)
    a1, a2 = _agg_call(g1, g2, src_agg, dst_agg, ew_bc, z2)
    a1, a2 = a1[:N], a2[:N]

    noise = jax.random.normal(jax.random.key(42), (n, H2), dtype=x.dtype)
    z, z_bf16 = _t3_call(a1, g1, a2, g2, dinv,
                         b1.reshape(1, H2), b2.reshape(1, H2), noise)

    adj_rec = _decoder(z_bf16)
    return (adj_rec, z)


# speedup vs baseline: 1.1586x; 1.1586x over previous
import functools

import jax
import jax.numpy as jnp
from jax import lax
from jax.experimental import pallas as pl
from jax.experimental.pallas import tpu as pltpu
from jax.experimental.pallas import tpu_sc as plsc

N = 10000
NPAD = 10240
E = 160000
EPAD = 163840
IN_DIM = 256
H1 = 256
H2 = 128

NC = 2
NS = 16
L = 16

ECHUNK = 128
CA = 64
DEG_NCHUNK = EPAD // (NC * NS * ECHUNK)
AGG_NCHUNK = EPAD // (NS * CA)
SLAB = NPAD // NS
WB_CHUNK = 128

_sc_mesh = plsc.VectorSubcoreMesh(core_axis_name="c", subcore_axis_name="s")


def _deg_body(dst_hbm, ew_hbm, z1_hbm, deg0_out, deg1_out,
              dst_v, ew_v, zb, shared_deg):
    c = lax.axis_index("c")
    s = lax.axis_index("s")
    wid = c * NS + s

    pltpu.sync_copy(dst_hbm.at[wid], dst_v)
    pltpu.sync_copy(ew_hbm.at[wid], ew_v)

    pltpu.sync_copy(z1_hbm, shared_deg.at[pl.ds(s * SLAB, SLAB)])
    plsc.subcore_barrier()

    def chunk_body(i, _):
        pltpu.sync_copy(ew_v.at[i], shared_deg.at[dst_v.at[i]], add=True)
        return 0
    lax.fori_loop(0, DEG_NCHUNK, chunk_body, 0)
    plsc.subcore_barrier()

    pltpu.sync_copy(shared_deg.at[pl.ds(s * SLAB, SLAB)], zb)

    @pl.when(c == 0)
    def _():
        pltpu.sync_copy(zb, deg0_out.at[pl.ds(s * SLAB, SLAB)])

    @pl.when(c == 1)
    def _():
        pltpu.sync_copy(zb, deg1_out.at[pl.ds(s * SLAB, SLAB)])


@functools.partial(
    pl.kernel,
    out_type=(jax.ShapeDtypeStruct((NPAD,), jnp.float32),
              jax.ShapeDtypeStruct((NPAD,), jnp.float32)),
    mesh=_sc_mesh,
    scratch_types=[
        pltpu.VMEM((DEG_NCHUNK, ECHUNK), jnp.int32),
        pltpu.VMEM((DEG_NCHUNK, ECHUNK), jnp.float32),
        pltpu.VMEM((SLAB,), jnp.float32),
        pltpu.VMEM_SHARED((NPAD,), jnp.float32),
    ],
)
def _deg_call(dst_hbm, ew_hbm, z1_hbm, deg0_out, deg1_out,
              dst_v, ew_v, zb, shared_deg):
    _deg_body(dst_hbm, ew_hbm, z1_hbm, deg0_out, deg1_out,
              dst_v, ew_v, zb, shared_deg)


def _agg_body(ga_hbm, gb_hbm, src_hbm, dst_hbm, ewb_hbm, z2_hbm,
              outa_hbm, outb_hbm,
              idx4, wrow2, rows2,
              semg_a, semg_b, sems_a, sems_b, semw_a, semw_b, semd_a, semd_b,
              acc):
    c = lax.axis_index("c")
    s = lax.axis_index("s")

    pltpu.sync_copy(z2_hbm, acc.at[pl.ds(s * SLAB, SLAB)])
    plsc.subcore_barrier()

    def scale_rows(p):
        @plsc.parallel_loop(0, CA, step=1, unroll=4)
        def _(r):
            row = p * CA + r
            w = wrow2[row, pl.ds(0, L)]
            for f in range(H2 // L):
                rows2[row, pl.ds(f * L, L)] = rows2[row, pl.ds(f * L, L)] * w

    def slot(p):
        return (idx4.at[2 * p], idx4.at[2 * p + 1],
                wrow2.at[pl.ds(p * CA, CA)],
                rows2.at[pl.ds(p * CA, CA)],
                (semg_a, semg_b)[p], (sems_a, sems_b)[p],
                (semw_a, semw_b)[p], (semd_a, semd_b)[p])

    def edge_loop(g_ref):
        def issue(i, p):
            src_c, dst_c, wrow, rows, semg, sems, semw, semd = slot(p)
            pltpu.async_copy(ewb_hbm.at[s, i], wrow, semw)
            pltpu.async_copy(dst_hbm.at[s, i], dst_c, semd)
            pltpu.sync_copy(src_hbm.at[s, i], src_c)
            pltpu.async_copy(g_ref.at[src_c], rows, semg)

        def process(i, p):
            src_c, dst_c, wrow, rows, semg, sems, semw, semd = slot(p)
            pltpu.make_async_copy(g_ref.at[src_c], rows, semg).wait()
            pltpu.make_async_copy(ewb_hbm.at[s, i], wrow, semw).wait()
            scale_rows(p)
            pltpu.make_async_copy(dst_hbm.at[s, i], dst_c, semd).wait()
            pltpu.async_copy(rows, acc.at[dst_c], sems, add=True)

        def wait_scatter(p):
            src_c, dst_c, wrow, rows, semg, sems, semw, semd = slot(p)
            pltpu.make_async_copy(rows, acc.at[dst_c], sems).wait()

        issue(0, 0)

        def body(j, _):
            i0 = 2 * j

            @pl.when(j > 0)
            def _():
                wait_scatter(1)
            issue(i0 + 1, 1)
            process(i0, 0)
            process(i0 + 1, 1)
            wait_scatter(0)
            issue((i0 + 2) % AGG_NCHUNK, 0)
            return 0
        lax.fori_loop(0, AGG_NCHUNK // 2, body, 0)

        wait_scatter(1)
        src_c, dst_c, wrow, rows, semg, sems, semw, semd = slot(0)
        pltpu.make_async_copy(g_ref.at[src_c], rows, semg).wait()
        pltpu.make_async_copy(ewb_hbm.at[s, 0], wrow, semw).wait()
        pltpu.make_async_copy(dst_hbm.at[s, 0], dst_c, semd).wait()

    @pl.when(c == 0)
    def _():
        edge_loop(ga_hbm)

    @pl.when(c == 1)
    def _():
        edge_loop(gb_hbm)

    plsc.subcore_barrier()

    def wb(out_ref):
        for i in range(SLAB // WB_CHUNK):
            base = s * SLAB + i * WB_CHUNK
            pltpu.sync_copy(acc.at[pl.ds(base, WB_CHUNK)], rows2)
            pltpu.sync_copy(rows2, out_ref.at[pl.ds(base, WB_CHUNK)])

    @pl.when(c == 0)
    def _():
        wb(outa_hbm)

    @pl.when(c == 1)
    def _():
        wb(outb_hbm)


@functools.partial(
    pl.kernel,
    out_type=(jax.ShapeDtypeStruct((NPAD, H2), jnp.float32),
              jax.ShapeDtypeStruct((NPAD, H2), jnp.float32)),
    mesh=_sc_mesh,
    scratch_types=[
        pltpu.VMEM((4, CA), jnp.int32),
        pltpu.VMEM((2 * CA, L), jnp.float32),
        pltpu.VMEM((2 * CA, H2), jnp.float32),
        pltpu.SemaphoreType.DMA,
        pltpu.SemaphoreType.DMA,
        pltpu.SemaphoreType.DMA,
        pltpu.SemaphoreType.DMA,
        pltpu.SemaphoreType.DMA,
        pltpu.SemaphoreType.DMA,
        pltpu.SemaphoreType.DMA,
        pltpu.SemaphoreType.DMA,
        pltpu.VMEM_SHARED((NPAD, H2), jnp.float32),
    ],
)
def _agg_call(ga_hbm, gb_hbm, src_hbm, dst_hbm, ewb_hbm, z2_hbm,
              outa_hbm, outb_hbm,
              idx4, wrow2, rows2,
              semg_a, semg_b, sems_a, sems_b, semw_a, semw_b, semd_a, semd_b,
              acc):
    _agg_body(ga_hbm, gb_hbm, src_hbm, dst_hbm, ewb_hbm, z2_hbm,
              outa_hbm, outb_hbm,
              idx4, wrow2, rows2,
              semg_a, semg_b, sems_a, sems_b, semw_a, semw_b, semd_a, semd_b,
              acc)


BR = 1000


def _dinv_body(degp_ref, dinv_ref):
    d = degp_ref[0, :] + degp_ref[1, :] + 1.0
    dinv_ref[...] = lax.rsqrt(d)[:, None]


def _dinv_call(deg_p):
    return pl.pallas_call(
        _dinv_body,
        out_shape=jax.ShapeDtypeStruct((NPAD, 1), jnp.float32),
    )(deg_p)


def _t1_body(x_ref, w_ref, dinv_ref, glo_ref, ghi_ref):
    h = jnp.dot(x_ref[...], w_ref[...], preferred_element_type=jnp.float32)
    g = h * dinv_ref[...]
    glo_ref[...] = g[:, :H2]
    ghi_ref[...] = g[:, H2:]


def _t1_call(x, W0, dinv):
    return pl.pallas_call(
        _t1_body,
        grid=(N // BR,),
        in_specs=[
            pl.BlockSpec((BR, IN_DIM), lambda i: (i, 0)),
            pl.BlockSpec((IN_DIM, H1), lambda i: (0, 0)),
            pl.BlockSpec((BR, 1), lambda i: (i, 0)),
        ],
        out_specs=(
            pl.BlockSpec((BR, H2), lambda i: (i, 0)),
            pl.BlockSpec((BR, H2), lambda i: (i, 0)),
        ),
        out_shape=(jax.ShapeDtypeStruct((N, H2), jnp.float32),
                   jax.ShapeDtypeStruct((N, H2), jnp.float32)),
    )(x, W0, dinv)


def _t2_body(alo_ref, ahi_ref, glo_ref, ghi_ref, dinv_ref, b0_ref, wc_ref,
             g1_ref, g2_ref):
    dinv = dinv_ref[...]
    hlo = (alo_ref[...] + glo_ref[...]) * dinv
    hhi = (ahi_ref[...] + ghi_ref[...]) * dinv
    h = jnp.concatenate([hlo, hhi], axis=1) + b0_ref[...]
    h = jnp.maximum(h, 0.0)
    m = jnp.dot(h, wc_ref[...], preferred_element_type=jnp.float32)
    g1_ref[...] = m[:, :H2] * dinv
    g2_ref[...] = m[:, H2:] * dinv


def _t2_call(alo, ahi, glo, ghi, dinv, b0, Wc):
    return pl.pallas_call(
        _t2_body,
        grid=(N // BR,),
        in_specs=[
            pl.BlockSpec((BR, H2), lambda i: (i, 0)),
            pl.BlockSpec((BR, H2), lambda i: (i, 0)),
            pl.BlockSpec((BR, H2), lambda i: (i, 0)),
            pl.BlockSpec((BR, H2), lambda i: (i, 0)),
            pl.BlockSpec((BR, 1), lambda i: (i, 0)),
            pl.BlockSpec((1, H1), lambda i: (0, 0)),
            pl.BlockSpec((H1, 2 * H2), lambda i: (0, 0)),
        ],
        out_specs=(
            pl.BlockSpec((BR, H2), lambda i: (i, 0)),
            pl.BlockSpec((BR, H2), lambda i: (i, 0)),
        ),
        out_shape=(jax.ShapeDtypeStruct((N, H2), jnp.float32),
                   jax.ShapeDtypeStruct((N, H2), jnp.float32)),
    )(alo, ahi, glo, ghi, dinv, b0, Wc)


def _t3_body(a1_ref, g1_ref, a2_ref, g2_ref, dinv_ref, b1_ref, b2_ref,
             noise_ref, z_ref, zb_ref):
    dinv = dinv_ref[...]
    mean = (a1_ref[...] + g1_ref[...]) * dinv + b1_ref[...]
    log_std = (a2_ref[...] + g2_ref[...]) * dinv + b2_ref[...]
    z = mean + noise_ref[...] * jnp.exp(log_std)
    z_ref[...] = z
    zb_ref[...] = z.astype(jnp.bfloat16)


def _t3_call(a1, g1, a2, g2, dinv, b1, b2, noise):
    return pl.pallas_call(
        _t3_body,
        grid=(N // BR,),
        in_specs=[
            pl.BlockSpec((BR, H2), lambda i: (i, 0)),
            pl.BlockSpec((BR, H2), lambda i: (i, 0)),
            pl.BlockSpec((BR, H2), lambda i: (i, 0)),
            pl.BlockSpec((BR, H2), lambda i: (i, 0)),
            pl.BlockSpec((BR, 1), lambda i: (i, 0)),
            pl.BlockSpec((1, H2), lambda i: (0, 0)),
            pl.BlockSpec((1, H2), lambda i: (0, 0)),
            pl.BlockSpec((BR, H2), lambda i: (i, 0)),
        ],
        out_specs=(pl.BlockSpec((BR, H2), lambda i: (i, 0)),
                   pl.BlockSpec((BR, H2), lambda i: (i, 0))),
        out_shape=(jax.ShapeDtypeStruct((N, H2), jnp.float32),
                   jax.ShapeDtypeStruct((N, H2), jnp.bfloat16)),
    )(a1, g1, a2, g2, dinv, b1, b2, noise)


DEC_BM = 200


def _decoder_body(zi_ref, zj_ref, out_ref):
    acc = jax.lax.dot_general(
        zi_ref[...], zj_ref[...],
        (((1,), (1,)), ((), ())),
        preferred_element_type=jnp.float32,
    )
    out_ref[...] = jax.nn.sigmoid(acc)


def _decoder(zb):
    n = zb.shape[0]
    return pl.pallas_call(
        _decoder_body,
        grid=(n // DEC_BM,),
        in_specs=[
            pl.BlockSpec((DEC_BM, H2), lambda i: (i, 0)),
            pl.BlockSpec((n, H2), lambda i: (0, 0)),
        ],
        out_specs=pl.BlockSpec((DEC_BM, n), lambda i: (i, 0)),
        out_shape=jax.ShapeDtypeStruct((n, n), jnp.float32),
    )(zb, zb)


def kernel(x, edge_index, edge_weight, W0, b0, W1, b1, W2, b2):
    src = edge_index[0]
    dst = edge_index[1]
    n = x.shape[0]

    pad = EPAD - E
    pad_idx = (jnp.arange(pad, dtype=jnp.int32) * 37) % N
    src_p = jnp.concatenate([src, pad_idx])
    dst_p = jnp.concatenate([dst, pad_idx])
    ew_p = jnp.concatenate([edge_weight, jnp.zeros((pad,), jnp.float32)])

    dst_deg = dst_p.reshape(NC * NS, DEG_NCHUNK, ECHUNK)
    ew_deg = ew_p.reshape(NC * NS, DEG_NCHUNK, ECHUNK)
    src_agg = src_p.reshape(NS, AGG_NCHUNK, CA)
    dst_agg = dst_p.reshape(NS, AGG_NCHUNK, CA)
    ew_bc = jnp.broadcast_to(
        ew_p[:, None], (EPAD, L)).reshape(NS, AGG_NCHUNK, CA, L)

    z1 = jnp.zeros((SLAB,), jnp.float32)
    z2 = jnp.zeros((SLAB, H2), jnp.float32)

    deg0, deg1 = _deg_call(dst_deg, ew_deg, z1)
    dinv_pad = _dinv_call(jnp.stack([deg0, deg1]))
    dinv = dinv_pad[:N]

    g0_lo, g0_hi = _t1_call(x, W0, dinv)
    a0_lo, a0_hi = _agg_call(g0_lo, g0_hi, src_agg, dst_agg, ew_bc, z2)
    a0_lo, a0_hi = a0_lo[:N], a0_hi[:N]

    Wc = jnp.concatenate([W1, W2], axis=1)
    g1, g2 = _t2_call(a0_lo, a0_hi, g0_lo, g0_hi, dinv,
                      b0.reshape(1, H1), Wc)
    a1, a2 = _agg_call(g1, g2, src_agg, dst_agg, ew_bc, z2)
    a1, a2 = a1[:N], a2[:N]

    noise = jax.random.normal(jax.random.key(42), (n, H2), dtype=x.dtype)
    z, z_bf16 = _t3_call(a1, g1, a2, g2, dinv,
                         b1.reshape(1, H2), b2.reshape(1, H2), noise)

    adj_rec = _decoder(z_bf16)
    return (adj_rec, z)


# speedup vs baseline: 1.1912x; 1.0281x over previous
import functools

import jax
import jax.numpy as jnp
from jax import lax
from jax.experimental import pallas as pl
from jax.experimental.pallas import tpu as pltpu
from jax.experimental.pallas import tpu_sc as plsc

N = 10000
NPAD = 10240
E = 160000
EPAD = 163840
IN_DIM = 256
H1 = 256
H2 = 128

NC = 2
NS = 16
L = 16

ECHUNK = 128
CA = 64
DEG_NCHUNK = EPAD // (NC * NS * ECHUNK)
AGG_NCHUNK = EPAD // (NS * CA)
SLAB = NPAD // NS
WB_CHUNK = 128

_sc_mesh = plsc.VectorSubcoreMesh(core_axis_name="c", subcore_axis_name="s")


def _deg_body(dst_hbm, ew_hbm, z1_hbm, deg0_out, deg1_out,
              dst_v, ew_v, zb, shared_deg):
    c = lax.axis_index("c")
    s = lax.axis_index("s")
    wid = c * NS + s

    pltpu.sync_copy(dst_hbm.at[wid], dst_v)
    pltpu.sync_copy(ew_hbm.at[wid], ew_v)

    pltpu.sync_copy(z1_hbm, shared_deg.at[pl.ds(s * SLAB, SLAB)])
    plsc.subcore_barrier()

    def chunk_body(i, _):
        pltpu.sync_copy(ew_v.at[i], shared_deg.at[dst_v.at[i]], add=True)
        return 0
    lax.fori_loop(0, DEG_NCHUNK, chunk_body, 0)
    plsc.subcore_barrier()

    pltpu.sync_copy(shared_deg.at[pl.ds(s * SLAB, SLAB)], zb)

    @pl.when(c == 0)
    def _():
        pltpu.sync_copy(zb, deg0_out.at[pl.ds(s * SLAB, SLAB)])

    @pl.when(c == 1)
    def _():
        pltpu.sync_copy(zb, deg1_out.at[pl.ds(s * SLAB, SLAB)])


@functools.partial(
    pl.kernel,
    out_type=(jax.ShapeDtypeStruct((NPAD,), jnp.float32),
              jax.ShapeDtypeStruct((NPAD,), jnp.float32)),
    mesh=_sc_mesh,
    scratch_types=[
        pltpu.VMEM((DEG_NCHUNK, ECHUNK), jnp.int32),
        pltpu.VMEM((DEG_NCHUNK, ECHUNK), jnp.float32),
        pltpu.VMEM((SLAB,), jnp.float32),
        pltpu.VMEM_SHARED((NPAD,), jnp.float32),
    ],
)
def _deg_call(dst_hbm, ew_hbm, z1_hbm, deg0_out, deg1_out,
              dst_v, ew_v, zb, shared_deg):
    _deg_body(dst_hbm, ew_hbm, z1_hbm, deg0_out, deg1_out,
              dst_v, ew_v, zb, shared_deg)


def _agg_body(ga_hbm, gb_hbm, src_hbm, dst_hbm, ewb_hbm, z2_hbm,
              outa_hbm, outb_hbm,
              idx4, wrow2, rows2,
              semg_a, semg_b, sems_a, sems_b, semw_a, semw_b, semd_a, semd_b,
              semi_a, semi_b,
              acc):
    c = lax.axis_index("c")
    s = lax.axis_index("s")

    pltpu.sync_copy(z2_hbm, acc.at[pl.ds(s * SLAB, SLAB)])
    plsc.subcore_barrier()

    def scale_rows(p):
        @plsc.parallel_loop(0, CA, step=1, unroll=4)
        def _(r):
            row = p * CA + r
            w = wrow2[row, pl.ds(0, L)]
            for f in range(H2 // L):
                rows2[row, pl.ds(f * L, L)] = rows2[row, pl.ds(f * L, L)] * w

    def slot(p):
        return (idx4.at[2 * p], idx4.at[2 * p + 1],
                wrow2.at[pl.ds(p * CA, CA)],
                rows2.at[pl.ds(p * CA, CA)],
                (semg_a, semg_b)[p], (sems_a, sems_b)[p],
                (semw_a, semw_b)[p], (semd_a, semd_b)[p],
                (semi_a, semi_b)[p])

    def edge_loop(g_ref):
        def wait_src(i, p):
            src_c = idx4.at[2 * p]
            pltpu.make_async_copy(src_hbm.at[s, i], src_c,
                                  (semi_a, semi_b)[p]).wait()

        def issue(i, p):
            src_c, dst_c, wrow, rows, semg, sems, semw, semd, semi = slot(p)
            pltpu.async_copy(ewb_hbm.at[s, i], wrow, semw)
            pltpu.async_copy(dst_hbm.at[s, i], dst_c, semd)
            pltpu.async_copy(g_ref.at[src_c], rows, semg)

        def process(i, p, nxt):
            src_c, dst_c, wrow, rows, semg, sems, semw, semd, semi = slot(p)
            pltpu.make_async_copy(g_ref.at[src_c], rows, semg).wait()
            pltpu.async_copy(src_hbm.at[s, nxt], src_c, semi)
            pltpu.make_async_copy(ewb_hbm.at[s, i], wrow, semw).wait()
            scale_rows(p)
            pltpu.make_async_copy(dst_hbm.at[s, i], dst_c, semd).wait()
            pltpu.async_copy(rows, acc.at[dst_c], sems, add=True)

        def wait_scatter(p):
            src_c, dst_c, wrow, rows, semg, sems, semw, semd, semi = slot(p)
            pltpu.make_async_copy(rows, acc.at[dst_c], sems).wait()

        pltpu.sync_copy(src_hbm.at[s, 0], idx4.at[0])
        pltpu.sync_copy(src_hbm.at[s, 1], idx4.at[2])
        issue(0, 0)

        def body(j, _):
            i0 = 2 * j

            @pl.when(j > 0)
            def _():
                wait_scatter(1)
                wait_src(i0 + 1, 1)
            issue(i0 + 1, 1)
            process(i0, 0, (i0 + 2) % AGG_NCHUNK)
            process(i0 + 1, 1, (i0 + 3) % AGG_NCHUNK)
            wait_scatter(0)
            wait_src((i0 + 2) % AGG_NCHUNK, 0)
            issue((i0 + 2) % AGG_NCHUNK, 0)
            return 0
        lax.fori_loop(0, AGG_NCHUNK // 2, body, 0)

        wait_scatter(1)
        wait_src(1 % AGG_NCHUNK, 1)
        src_c, dst_c, wrow, rows, semg, sems, semw, semd, semi = slot(0)
        pltpu.make_async_copy(g_ref.at[src_c], rows, semg).wait()
        pltpu.make_async_copy(ewb_hbm.at[s, 0], wrow, semw).wait()
        pltpu.make_async_copy(dst_hbm.at[s, 0], dst_c, semd).wait()

    @pl.when(c == 0)
    def _():
        edge_loop(ga_hbm)

    @pl.when(c == 1)
    def _():
        edge_loop(gb_hbm)

    plsc.subcore_barrier()

    def wb(out_ref):
        for i in range(SLAB // WB_CHUNK):
            base = s * SLAB + i * WB_CHUNK
            pltpu.sync_copy(acc.at[pl.ds(base, WB_CHUNK)], rows2)
            pltpu.sync_copy(rows2, out_ref.at[pl.ds(base, WB_CHUNK)])

    @pl.when(c == 0)
    def _():
        wb(outa_hbm)

    @pl.when(c == 1)
    def _():
        wb(outb_hbm)


@functools.partial(
    pl.kernel,
    out_type=(jax.ShapeDtypeStruct((NPAD, H2), jnp.float32),
              jax.ShapeDtypeStruct((NPAD, H2), jnp.float32)),
    mesh=_sc_mesh,
    scratch_types=[
        pltpu.VMEM((4, CA), jnp.int32),
        pltpu.VMEM((2 * CA, L), jnp.float32),
        pltpu.VMEM((2 * CA, H2), jnp.float32),
        pltpu.SemaphoreType.DMA,
        pltpu.SemaphoreType.DMA,
        pltpu.SemaphoreType.DMA,
        pltpu.SemaphoreType.DMA,
        pltpu.SemaphoreType.DMA,
        pltpu.SemaphoreType.DMA,
        pltpu.SemaphoreType.DMA,
        pltpu.SemaphoreType.DMA,
        pltpu.SemaphoreType.DMA,
        pltpu.SemaphoreType.DMA,
        pltpu.VMEM_SHARED((NPAD, H2), jnp.float32),
    ],
)
def _agg_call(ga_hbm, gb_hbm, src_hbm, dst_hbm, ewb_hbm, z2_hbm,
              outa_hbm, outb_hbm,
              idx4, wrow2, rows2,
              semg_a, semg_b, sems_a, sems_b, semw_a, semw_b, semd_a, semd_b,
              semi_a, semi_b,
              acc):
    _agg_body(ga_hbm, gb_hbm, src_hbm, dst_hbm, ewb_hbm, z2_hbm,
              outa_hbm, outb_hbm,
              idx4, wrow2, rows2,
              semg_a, semg_b, sems_a, sems_b, semw_a, semw_b, semd_a, semd_b,
              semi_a, semi_b,
              acc)


BR = 1000


def _dinv_body(degp_ref, dinv_ref):
    d = degp_ref[0, :] + degp_ref[1, :] + 1.0
    dinv_ref[...] = lax.rsqrt(d)[:, None]


def _dinv_call(deg_p):
    return pl.pallas_call(
        _dinv_body,
        out_shape=jax.ShapeDtypeStruct((NPAD, 1), jnp.float32),
    )(deg_p)


def _t1_body(x_ref, w_ref, dinv_ref, glo_ref, ghi_ref):
    h = jnp.dot(x_ref[...], w_ref[...], preferred_element_type=jnp.float32)
    g = h * dinv_ref[...]
    glo_ref[...] = g[:, :H2]
    ghi_ref[...] = g[:, H2:]


def _t1_call(x, W0, dinv):
    return pl.pallas_call(
        _t1_body,
        grid=(N // BR,),
        in_specs=[
            pl.BlockSpec((BR, IN_DIM), lambda i: (i, 0)),
            pl.BlockSpec((IN_DIM, H1), lambda i: (0, 0)),
            pl.BlockSpec((BR, 1), lambda i: (i, 0)),
        ],
        out_specs=(
            pl.BlockSpec((BR, H2), lambda i: (i, 0)),
            pl.BlockSpec((BR, H2), lambda i: (i, 0)),
        ),
        out_shape=(jax.ShapeDtypeStruct((N, H2), jnp.float32),
                   jax.ShapeDtypeStruct((N, H2), jnp.float32)),
    )(x, W0, dinv)


def _t2_body(alo_ref, ahi_ref, glo_ref, ghi_ref, dinv_ref, b0_ref, wc_ref,
             g1_ref, g2_ref):
    dinv = dinv_ref[...]
    hlo = (alo_ref[...] + glo_ref[...]) * dinv
    hhi = (ahi_ref[...] + ghi_ref[...]) * dinv
    h = jnp.concatenate([hlo, hhi], axis=1) + b0_ref[...]
    h = jnp.maximum(h, 0.0)
    m = jnp.dot(h, wc_ref[...], preferred_element_type=jnp.float32)
    g1_ref[...] = m[:, :H2] * dinv
    g2_ref[...] = m[:, H2:] * dinv


def _t2_call(alo, ahi, glo, ghi, dinv, b0, Wc):
    return pl.pallas_call(
        _t2_body,
        grid=(N // BR,),
        in_specs=[
            pl.BlockSpec((BR, H2), lambda i: (i, 0)),
            pl.BlockSpec((BR, H2), lambda i: (i, 0)),
            pl.BlockSpec((BR, H2), lambda i: (i, 0)),
            pl.BlockSpec((BR, H2), lambda i: (i, 0)),
            pl.BlockSpec((BR, 1), lambda i: (i, 0)),
            pl.BlockSpec((1, H1), lambda i: (0, 0)),
            pl.BlockSpec((H1, 2 * H2), lambda i: (0, 0)),
        ],
        out_specs=(
            pl.BlockSpec((BR, H2), lambda i: (i, 0)),
            pl.BlockSpec((BR, H2), lambda i: (i, 0)),
        ),
        out_shape=(jax.ShapeDtypeStruct((N, H2), jnp.float32),
                   jax.ShapeDtypeStruct((N, H2), jnp.float32)),
    )(alo, ahi, glo, ghi, dinv, b0, Wc)


def _t3_body(a1_ref, g1_ref, a2_ref, g2_ref, dinv_ref, b1_ref, b2_ref,
             noise_ref, z_ref, zb_ref):
    dinv = dinv_ref[...]
    mean = (a1_ref[...] + g1_ref[...]) * dinv + b1_ref[...]
    log_std = (a2_ref[...] + g2_ref[...]) * dinv + b2_ref[...]
    z = mean + noise_ref[...] * jnp.exp(log_std)
    z_ref[...] = z
    zb_ref[...] = z.astype(jnp.bfloat16)


def _t3_call(a1, g1, a2, g2, dinv, b1, b2, noise):
    return pl.pallas_call(
        _t3_body,
        grid=(N // BR,),
        in_specs=[
            pl.BlockSpec((BR, H2), lambda i: (i, 0)),
            pl.BlockSpec((BR, H2), lambda i: (i, 0)),
            pl.BlockSpec((BR, H2), lambda i: (i, 0)),
            pl.BlockSpec((BR, H2), lambda i: (i, 0)),
            pl.BlockSpec((BR, 1), lambda i: (i, 0)),
            pl.BlockSpec((1, H2), lambda i: (0, 0)),
            pl.BlockSpec((1, H2), lambda i: (0, 0)),
            pl.BlockSpec((BR, H2), lambda i: (i, 0)),
        ],
        out_specs=(pl.BlockSpec((BR, H2), lambda i: (i, 0)),
                   pl.BlockSpec((BR, H2), lambda i: (i, 0))),
        out_shape=(jax.ShapeDtypeStruct((N, H2), jnp.float32),
                   jax.ShapeDtypeStruct((N, H2), jnp.bfloat16)),
    )(a1, g1, a2, g2, dinv, b1, b2, noise)


DEC_BM = 200


def _decoder_body(zi_ref, zj_ref, out_ref):
    acc = jax.lax.dot_general(
        zi_ref[...], zj_ref[...],
        (((1,), (1,)), ((), ())),
        preferred_element_type=jnp.float32,
    )
    out_ref[...] = jax.nn.sigmoid(acc)


def _decoder(zb):
    n = zb.shape[0]
    return pl.pallas_call(
        _decoder_body,
        grid=(n // DEC_BM,),
        in_specs=[
            pl.BlockSpec((DEC_BM, H2), lambda i: (i, 0)),
            pl.BlockSpec((n, H2), lambda i: (0, 0)),
        ],
        out_specs=pl.BlockSpec((DEC_BM, n), lambda i: (i, 0)),
        out_shape=jax.ShapeDtypeStruct((n, n), jnp.float32),
    )(zb, zb)


def kernel(x, edge_index, edge_weight, W0, b0, W1, b1, W2, b2):
    src = edge_index[0]
    dst = edge_index[1]
    n = x.shape[0]

    pad = EPAD - E
    pad_idx = (jnp.arange(pad, dtype=jnp.int32) * 37) % N
    src_p = jnp.concatenate([src, pad_idx])
    dst_p = jnp.concatenate([dst, pad_idx])
    ew_p = jnp.concatenate([edge_weight, jnp.zeros((pad,), jnp.float32)])

    dst_deg = dst_p.reshape(NC * NS, DEG_NCHUNK, ECHUNK)
    ew_deg = ew_p.reshape(NC * NS, DEG_NCHUNK, ECHUNK)
    src_agg = src_p.reshape(NS, AGG_NCHUNK, CA)
    dst_agg = dst_p.reshape(NS, AGG_NCHUNK, CA)
    ew_bc = jnp.broadcast_to(
        ew_p[:, None], (EPAD, L)).reshape(NS, AGG_NCHUNK, CA, L)

    z1 = jnp.zeros((SLAB,), jnp.float32)
    z2 = jnp.zeros((SLAB, H2), jnp.float32)

    deg0, deg1 = _deg_call(dst_deg, ew_deg, z1)
    dinv_pad = _dinv_call(jnp.stack([deg0, deg1]))
    dinv = dinv_pad[:N]

    g0_lo, g0_hi = _t1_call(x, W0, dinv)
    a0_lo, a0_hi = _agg_call(g0_lo, g0_hi, src_agg, dst_agg, ew_bc, z2)
    a0_lo, a0_hi = a0_lo[:N], a0_hi[:N]

    Wc = jnp.concatenate([W1, W2], axis=1)
    g1, g2 = _t2_call(a0_lo, a0_hi, g0_lo, g0_hi, dinv,
                      b0.reshape(1, H1), Wc)
    a1, a2 = _agg_call(g1, g2, src_agg, dst_agg, ew_bc, z2)
    a1, a2 = a1[:N], a2[:N]

    noise = jax.random.normal(jax.random.key(42), (n, H2), dtype=x.dtype)
    z, z_bf16 = _t3_call(a1, g1, a2, g2, dinv,
                         b1.reshape(1, H2), b2.reshape(1, H2), noise)

    adj_rec = _decoder(z_bf16)
    return (adj_rec, z)


# speedup vs baseline: 1.2151x; 1.0201x over previous
import functools

import jax
import jax.numpy as jnp
from jax import lax
from jax.experimental import pallas as pl
from jax.experimental.pallas import tpu as pltpu
from jax.experimental.pallas import tpu_sc as plsc

N = 10000
NPAD = 10240
E = 160000
EPAD = 163840
IN_DIM = 256
H1 = 256
H2 = 128

NC = 2
NS = 16
L = 16

ECHUNK = 128
CA = 80
DEG_NCHUNK = EPAD // (NC * NS * ECHUNK)
AGG_NCHUNK = EPAD // (NS * CA)
SLAB = NPAD // NS
WB_CHUNK = 128

_sc_mesh = plsc.VectorSubcoreMesh(core_axis_name="c", subcore_axis_name="s")


def _deg_body(dst_hbm, ew_hbm, z1_hbm, deg0_out, deg1_out,
              dst_v, ew_v, zb, shared_deg):
    c = lax.axis_index("c")
    s = lax.axis_index("s")
    wid = c * NS + s

    pltpu.sync_copy(dst_hbm.at[wid], dst_v)
    pltpu.sync_copy(ew_hbm.at[wid], ew_v)

    pltpu.sync_copy(z1_hbm, shared_deg.at[pl.ds(s * SLAB, SLAB)])
    plsc.subcore_barrier()

    def chunk_body(i, _):
        pltpu.sync_copy(ew_v.at[i], shared_deg.at[dst_v.at[i]], add=True)
        return 0
    lax.fori_loop(0, DEG_NCHUNK, chunk_body, 0)
    plsc.subcore_barrier()

    pltpu.sync_copy(shared_deg.at[pl.ds(s * SLAB, SLAB)], zb)

    @pl.when(c == 0)
    def _():
        pltpu.sync_copy(zb, deg0_out.at[pl.ds(s * SLAB, SLAB)])

    @pl.when(c == 1)
    def _():
        pltpu.sync_copy(zb, deg1_out.at[pl.ds(s * SLAB, SLAB)])


@functools.partial(
    pl.kernel,
    out_type=(jax.ShapeDtypeStruct((NPAD,), jnp.float32),
              jax.ShapeDtypeStruct((NPAD,), jnp.float32)),
    mesh=_sc_mesh,
    scratch_types=[
        pltpu.VMEM((DEG_NCHUNK, ECHUNK), jnp.int32),
        pltpu.VMEM((DEG_NCHUNK, ECHUNK), jnp.float32),
        pltpu.VMEM((SLAB,), jnp.float32),
        pltpu.VMEM_SHARED((NPAD,), jnp.float32),
    ],
)
def _deg_call(dst_hbm, ew_hbm, z1_hbm, deg0_out, deg1_out,
              dst_v, ew_v, zb, shared_deg):
    _deg_body(dst_hbm, ew_hbm, z1_hbm, deg0_out, deg1_out,
              dst_v, ew_v, zb, shared_deg)


def _agg_body(ga_hbm, gb_hbm, src_hbm, dst_hbm, ewb_hbm, z2_hbm,
              outa_hbm, outb_hbm,
              idx4, wrow2, rows2,
              semg_a, semg_b, sems_a, sems_b, semw_a, semw_b, semd_a, semd_b,
              semi_a, semi_b,
              acc):
    c = lax.axis_index("c")
    s = lax.axis_index("s")

    pltpu.sync_copy(z2_hbm, acc.at[pl.ds(s * SLAB, SLAB)])
    plsc.subcore_barrier()

    def scale_rows(p):
        @plsc.parallel_loop(0, CA, step=1, unroll=4)
        def _(r):
            row = p * CA + r
            w = wrow2[row, pl.ds(0, L)]
            for f in range(H2 // L):
                rows2[row, pl.ds(f * L, L)] = rows2[row, pl.ds(f * L, L)] * w

    def slot(p):
        return (idx4.at[2 * p], idx4.at[2 * p + 1],
                wrow2.at[pl.ds(p * CA, CA)],
                rows2.at[pl.ds(p * CA, CA)],
                (semg_a, semg_b)[p], (sems_a, sems_b)[p],
                (semw_a, semw_b)[p], (semd_a, semd_b)[p],
                (semi_a, semi_b)[p])

    def edge_loop(g_ref):
        def wait_src(i, p):
            src_c = idx4.at[2 * p]
            pltpu.make_async_copy(src_hbm.at[s, i], src_c,
                                  (semi_a, semi_b)[p]).wait()

        def issue(i, p):
            src_c, dst_c, wrow, rows, semg, sems, semw, semd, semi = slot(p)
            pltpu.async_copy(ewb_hbm.at[s, i], wrow, semw)
            pltpu.async_copy(dst_hbm.at[s, i], dst_c, semd)
            pltpu.async_copy(g_ref.at[src_c], rows, semg)

        def process(i, p, nxt):
            src_c, dst_c, wrow, rows, semg, sems, semw, semd, semi = slot(p)
            pltpu.make_async_copy(g_ref.at[src_c], rows, semg).wait()
            pltpu.async_copy(src_hbm.at[s, nxt], src_c, semi)
            pltpu.make_async_copy(ewb_hbm.at[s, i], wrow, semw).wait()
            scale_rows(p)
            pltpu.make_async_copy(dst_hbm.at[s, i], dst_c, semd).wait()
            pltpu.async_copy(rows, acc.at[dst_c], sems, add=True)

        def wait_scatter(p):
            src_c, dst_c, wrow, rows, semg, sems, semw, semd, semi = slot(p)
            pltpu.make_async_copy(rows, acc.at[dst_c], sems).wait()

        pltpu.sync_copy(src_hbm.at[s, 0], idx4.at[0])
        pltpu.sync_copy(src_hbm.at[s, 1], idx4.at[2])
        issue(0, 0)

        def body(j, _):
            i0 = 2 * j

            @pl.when(j > 0)
            def _():
                wait_scatter(1)
                wait_src(i0 + 1, 1)
            issue(i0 + 1, 1)
            process(i0, 0, (i0 + 2) % AGG_NCHUNK)
            process(i0 + 1, 1, (i0 + 3) % AGG_NCHUNK)
            wait_scatter(0)
            wait_src((i0 + 2) % AGG_NCHUNK, 0)
            issue((i0 + 2) % AGG_NCHUNK, 0)
            return 0
        lax.fori_loop(0, AGG_NCHUNK // 2, body, 0)

        wait_scatter(1)
        wait_src(1 % AGG_NCHUNK, 1)
        src_c, dst_c, wrow, rows, semg, sems, semw, semd, semi = slot(0)
        pltpu.make_async_copy(g_ref.at[src_c], rows, semg).wait()
        pltpu.make_async_copy(ewb_hbm.at[s, 0], wrow, semw).wait()
        pltpu.make_async_copy(dst_hbm.at[s, 0], dst_c, semd).wait()

    @pl.when(c == 0)
    def _():
        edge_loop(ga_hbm)

    @pl.when(c == 1)
    def _():
        edge_loop(gb_hbm)

    plsc.subcore_barrier()

    def wb(out_ref):
        stage = rows2.at[pl.ds(0, WB_CHUNK)]
        for i in range(SLAB // WB_CHUNK):
            base = s * SLAB + i * WB_CHUNK
            pltpu.sync_copy(acc.at[pl.ds(base, WB_CHUNK)], stage)
            pltpu.sync_copy(stage, out_ref.at[pl.ds(base, WB_CHUNK)])

    @pl.when(c == 0)
    def _():
        wb(outa_hbm)

    @pl.when(c == 1)
    def _():
        wb(outb_hbm)


@functools.partial(
    pl.kernel,
    out_type=(jax.ShapeDtypeStruct((NPAD, H2), jnp.float32),
              jax.ShapeDtypeStruct((NPAD, H2), jnp.float32)),
    mesh=_sc_mesh,
    scratch_types=[
        pltpu.VMEM((4, CA), jnp.int32),
        pltpu.VMEM((2 * CA, L), jnp.float32),
        pltpu.VMEM((2 * CA, H2), jnp.float32),
        pltpu.SemaphoreType.DMA,
        pltpu.SemaphoreType.DMA,
        pltpu.SemaphoreType.DMA,
        pltpu.SemaphoreType.DMA,
        pltpu.SemaphoreType.DMA,
        pltpu.SemaphoreType.DMA,
        pltpu.SemaphoreType.DMA,
        pltpu.SemaphoreType.DMA,
        pltpu.SemaphoreType.DMA,
        pltpu.SemaphoreType.DMA,
        pltpu.VMEM_SHARED((NPAD, H2), jnp.float32),
    ],
)
def _agg_call(ga_hbm, gb_hbm, src_hbm, dst_hbm, ewb_hbm, z2_hbm,
              outa_hbm, outb_hbm,
              idx4, wrow2, rows2,
              semg_a, semg_b, sems_a, sems_b, semw_a, semw_b, semd_a, semd_b,
              semi_a, semi_b,
              acc):
    _agg_body(ga_hbm, gb_hbm, src_hbm, dst_hbm, ewb_hbm, z2_hbm,
              outa_hbm, outb_hbm,
              idx4, wrow2, rows2,
              semg_a, semg_b, sems_a, sems_b, semw_a, semw_b, semd_a, semd_b,
              semi_a, semi_b,
              acc)


BR = 1000


def _dinv_body(degp_ref, dinv_ref):
    d = degp_ref[0, :] + degp_ref[1, :] + 1.0
    dinv_ref[...] = lax.rsqrt(d)[:, None]


def _dinv_call(deg_p):
    return pl.pallas_call(
        _dinv_body,
        out_shape=jax.ShapeDtypeStruct((NPAD, 1), jnp.float32),
    )(deg_p)


def _t1_body(x_ref, w_ref, dinv_ref, glo_ref, ghi_ref):
    h = jnp.dot(x_ref[...], w_ref[...], preferred_element_type=jnp.float32)
    g = h * dinv_ref[...]
    glo_ref[...] = g[:, :H2]
    ghi_ref[...] = g[:, H2:]


def _t1_call(x, W0, dinv):
    return pl.pallas_call(
        _t1_body,
        grid=(N // BR,),
        in_specs=[
            pl.BlockSpec((BR, IN_DIM), lambda i: (i, 0)),
            pl.BlockSpec((IN_DIM, H1), lambda i: (0, 0)),
            pl.BlockSpec((BR, 1), lambda i: (i, 0)),
        ],
        out_specs=(
            pl.BlockSpec((BR, H2), lambda i: (i, 0)),
            pl.BlockSpec((BR, H2), lambda i: (i, 0)),
        ),
        out_shape=(jax.ShapeDtypeStruct((N, H2), jnp.float32),
                   jax.ShapeDtypeStruct((N, H2), jnp.float32)),
    )(x, W0, dinv)


def _t2_body(alo_ref, ahi_ref, glo_ref, ghi_ref, dinv_ref, b0_ref, wc_ref,
             g1_ref, g2_ref):
    dinv = dinv_ref[...]
    hlo = (alo_ref[...] + glo_ref[...]) * dinv
    hhi = (ahi_ref[...] + ghi_ref[...]) * dinv
    h = jnp.concatenate([hlo, hhi], axis=1) + b0_ref[...]
    h = jnp.maximum(h, 0.0)
    m = jnp.dot(h, wc_ref[...], preferred_element_type=jnp.float32)
    g1_ref[...] = m[:, :H2] * dinv
    g2_ref[...] = m[:, H2:] * dinv


def _t2_call(alo, ahi, glo, ghi, dinv, b0, Wc):
    return pl.pallas_call(
        _t2_body,
        grid=(N // BR,),
        in_specs=[
            pl.BlockSpec((BR, H2), lambda i: (i, 0)),
            pl.BlockSpec((BR, H2), lambda i: (i, 0)),
            pl.BlockSpec((BR, H2), lambda i: (i, 0)),
            pl.BlockSpec((BR, H2), lambda i: (i, 0)),
            pl.BlockSpec((BR, 1), lambda i: (i, 0)),
            pl.BlockSpec((1, H1), lambda i: (0, 0)),
            pl.BlockSpec((H1, 2 * H2), lambda i: (0, 0)),
        ],
        out_specs=(
            pl.BlockSpec((BR, H2), lambda i: (i, 0)),
            pl.BlockSpec((BR, H2), lambda i: (i, 0)),
        ),
        out_shape=(jax.ShapeDtypeStruct((N, H2), jnp.float32),
                   jax.ShapeDtypeStruct((N, H2), jnp.float32)),
    )(alo, ahi, glo, ghi, dinv, b0, Wc)


def _t3_body(a1_ref, g1_ref, a2_ref, g2_ref, dinv_ref, b1_ref, b2_ref,
             noise_ref, z_ref, zb_ref):
    dinv = dinv_ref[...]
    mean = (a1_ref[...] + g1_ref[...]) * dinv + b1_ref[...]
    log_std = (a2_ref[...] + g2_ref[...]) * dinv + b2_ref[...]
    z = mean + noise_ref[...] * jnp.exp(log_std)
    z_ref[...] = z
    zb_ref[...] = z.astype(jnp.bfloat16)


def _t3_call(a1, g1, a2, g2, dinv, b1, b2, noise):
    return pl.pallas_call(
        _t3_body,
        grid=(N // BR,),
        in_specs=[
            pl.BlockSpec((BR, H2), lambda i: (i, 0)),
            pl.BlockSpec((BR, H2), lambda i: (i, 0)),
            pl.BlockSpec((BR, H2), lambda i: (i, 0)),
            pl.BlockSpec((BR, H2), lambda i: (i, 0)),
            pl.BlockSpec((BR, 1), lambda i: (i, 0)),
            pl.BlockSpec((1, H2), lambda i: (0, 0)),
            pl.BlockSpec((1, H2), lambda i: (0, 0)),
            pl.BlockSpec((BR, H2), lambda i: (i, 0)),
        ],
        out_specs=(pl.BlockSpec((BR, H2), lambda i: (i, 0)),
                   pl.BlockSpec((BR, H2), lambda i: (i, 0))),
        out_shape=(jax.ShapeDtypeStruct((N, H2), jnp.float32),
                   jax.ShapeDtypeStruct((N, H2), jnp.bfloat16)),
    )(a1, g1, a2, g2, dinv, b1, b2, noise)


DEC_BM = 200


def _decoder_body(zi_ref, zj_ref, out_ref):
    acc = jax.lax.dot_general(
        zi_ref[...], zj_ref[...],
        (((1,), (1,)), ((), ())),
        preferred_element_type=jnp.float32,
    )
    out_ref[...] = jax.nn.sigmoid(acc)


def _decoder(zb):
    n = zb.shape[0]
    return pl.pallas_call(
        _decoder_body,
        grid=(n // DEC_BM,),
        in_specs=[
            pl.BlockSpec((DEC_BM, H2), lambda i: (i, 0)),
            pl.BlockSpec((n, H2), lambda i: (0, 0)),
        ],
        out_specs=pl.BlockSpec((DEC_BM, n), lambda i: (i, 0)),
        out_shape=jax.ShapeDtypeStruct((n, n), jnp.float32),
    )(zb, zb)


def kernel(x, edge_index, edge_weight, W0, b0, W1, b1, W2, b2):
    src = edge_index[0]
    dst = edge_index[1]
    n = x.shape[0]

    pad = EPAD - E
    pad_idx = (jnp.arange(pad, dtype=jnp.int32) * 37) % N
    src_p = jnp.concatenate([src, pad_idx])
    dst_p = jnp.concatenate([dst, pad_idx])
    ew_p = jnp.concatenate([edge_weight, jnp.zeros((pad,), jnp.float32)])

    dst_deg = dst_p.reshape(NC * NS, DEG_NCHUNK, ECHUNK)
    ew_deg = ew_p.reshape(NC * NS, DEG_NCHUNK, ECHUNK)
    src_agg = src_p.reshape(NS, AGG_NCHUNK, CA)
    dst_agg = dst_p.reshape(NS, AGG_NCHUNK, CA)
    ew_bc = jnp.broadcast_to(
        ew_p[:, None], (EPAD, L)).reshape(NS, AGG_NCHUNK, CA, L)

    z1 = jnp.zeros((SLAB,), jnp.float32)
    z2 = jnp.zeros((SLAB, H2), jnp.float32)

    deg0, deg1 = _deg_call(dst_deg, ew_deg, z1)
    dinv_pad = _dinv_call(jnp.stack([deg0, deg1]))
    dinv = dinv_pad[:N]

    g0_lo, g0_hi = _t1_call(x, W0, dinv)
    a0_lo, a0_hi = _agg_call(g0_lo, g0_hi, src_agg, dst_agg, ew_bc, z2)
    a0_lo, a0_hi = a0_lo[:N], a0_hi[:N]

    Wc = jnp.concatenate([W1, W2], axis=1)
    g1, g2 = _t2_call(a0_lo, a0_hi, g0_lo, g0_hi, dinv,
                      b0.reshape(1, H1), Wc)
    a1, a2 = _agg_call(g1, g2, src_agg, dst_agg, ew_bc, z2)
    a1, a2 = a1[:N], a2[:N]

    noise = jax.random.normal(jax.random.key(42), (n, H2), dtype=x.dtype)
    z, z_bf16 = _t3_call(a1, g1, a2, g2, dinv,
                         b1.reshape(1, H2), b2.reshape(1, H2), noise)

    adj_rec = _decoder(z_bf16)
    return (adj_rec, z)


# speedup vs baseline: 1.2391x; 1.0198x over previous
import functools

import jax
import jax.numpy as jnp
from jax import lax
from jax.experimental import pallas as pl
from jax.experimental.pallas import tpu as pltpu
from jax.experimental.pallas import tpu_sc as plsc

N = 10000
NPAD = 10240
E = 160000
EPAD = 163840
IN_DIM = 256
H1 = 256
H2 = 128

NC = 2
NS = 16
L = 16

ECHUNK = 128
CA = 80
DEG_NCHUNK = EPAD // (NC * NS * ECHUNK)
AGG_NCHUNK = EPAD // (NS * CA)
SLAB = NPAD // NS
WB_CHUNK = 128

_sc_mesh = plsc.VectorSubcoreMesh(core_axis_name="c", subcore_axis_name="s")


def _deg_body(dst_hbm, ew_hbm, z1_hbm, deg0_out, deg1_out,
              dst_v, ew_v, zb, shared_deg):
    c = lax.axis_index("c")
    s = lax.axis_index("s")
    wid = c * NS + s

    pltpu.sync_copy(dst_hbm.at[wid], dst_v)
    pltpu.sync_copy(ew_hbm.at[wid], ew_v)

    pltpu.sync_copy(z1_hbm, shared_deg.at[pl.ds(s * SLAB, SLAB)])
    plsc.subcore_barrier()

    def chunk_body(i, _):
        pltpu.sync_copy(ew_v.at[i], shared_deg.at[dst_v.at[i]], add=True)
        return 0
    lax.fori_loop(0, DEG_NCHUNK, chunk_body, 0)
    plsc.subcore_barrier()

    pltpu.sync_copy(shared_deg.at[pl.ds(s * SLAB, SLAB)], zb)

    @pl.when(c == 0)
    def _():
        pltpu.sync_copy(zb, deg0_out.at[pl.ds(s * SLAB, SLAB)])

    @pl.when(c == 1)
    def _():
        pltpu.sync_copy(zb, deg1_out.at[pl.ds(s * SLAB, SLAB)])


@functools.partial(
    pl.kernel,
    out_type=(jax.ShapeDtypeStruct((NPAD,), jnp.float32),
              jax.ShapeDtypeStruct((NPAD,), jnp.float32)),
    mesh=_sc_mesh,
    scratch_types=[
        pltpu.VMEM((DEG_NCHUNK, ECHUNK), jnp.int32),
        pltpu.VMEM((DEG_NCHUNK, ECHUNK), jnp.float32),
        pltpu.VMEM((SLAB,), jnp.float32),
        pltpu.VMEM_SHARED((NPAD,), jnp.float32),
    ],
)
def _deg_call(dst_hbm, ew_hbm, z1_hbm, deg0_out, deg1_out,
              dst_v, ew_v, zb, shared_deg):
    _deg_body(dst_hbm, ew_hbm, z1_hbm, deg0_out, deg1_out,
              dst_v, ew_v, zb, shared_deg)


def _agg_body(ga_hbm, gb_hbm, src_hbm, dst_hbm, ewb_hbm, z2_hbm,
              outa_hbm, outb_hbm,
              idx4, wrow2, rows2,
              semg_a, semg_b, sems_a, sems_b, semw_a, semw_b, semd_a, semd_b,
              semi_a, semi_b,
              acc):
    c = lax.axis_index("c")
    s = lax.axis_index("s")

    pltpu.sync_copy(z2_hbm, acc.at[pl.ds(s * SLAB, SLAB)])
    plsc.subcore_barrier()

    def scale_rows(p):
        @plsc.parallel_loop(0, CA, step=1, unroll=4)
        def _(r):
            row = p * CA + r
            w = wrow2[row, pl.ds(0, L)]
            for f in range(H2 // L):
                rows2[row, pl.ds(f * L, L)] = rows2[row, pl.ds(f * L, L)] * w

    def slot(p):
        return (idx4.at[2 * p], idx4.at[2 * p + 1],
                wrow2.at[pl.ds(p * CA, CA)],
                rows2.at[pl.ds(p * CA, CA)],
                (semg_a, semg_b)[p], (sems_a, sems_b)[p],
                (semw_a, semw_b)[p], (semd_a, semd_b)[p],
                (semi_a, semi_b)[p])

    def edge_loop(g_ref):
        def wait_src(i, p):
            src_c = idx4.at[2 * p]
            pltpu.make_async_copy(src_hbm.at[s, i], src_c,
                                  (semi_a, semi_b)[p]).wait()

        def issue(i, p):
            src_c, dst_c, wrow, rows, semg, sems, semw, semd, semi = slot(p)
            pltpu.async_copy(ewb_hbm.at[s, i], wrow, semw)
            pltpu.async_copy(dst_hbm.at[s, i], dst_c, semd)
            pltpu.async_copy(g_ref.at[src_c], rows, semg)

        def process(i, p, nxt):
            src_c, dst_c, wrow, rows, semg, sems, semw, semd, semi = slot(p)
            pltpu.make_async_copy(g_ref.at[src_c], rows, semg).wait()
            pltpu.async_copy(src_hbm.at[s, nxt], src_c, semi)
            pltpu.make_async_copy(ewb_hbm.at[s, i], wrow, semw).wait()
            scale_rows(p)
            pltpu.make_async_copy(dst_hbm.at[s, i], dst_c, semd).wait()
            pltpu.async_copy(rows, acc.at[dst_c], sems, add=True)

        def wait_scatter(p):
            src_c, dst_c, wrow, rows, semg, sems, semw, semd, semi = slot(p)
            pltpu.make_async_copy(rows, acc.at[dst_c], sems).wait()

        pltpu.sync_copy(src_hbm.at[s, 0], idx4.at[0])
        pltpu.sync_copy(src_hbm.at[s, 1], idx4.at[2])
        issue(0, 0)

        def body(j, _):
            i0 = 2 * j

            @pl.when(j > 0)
            def _():
                wait_scatter(1)
                wait_src(i0 + 1, 1)
            issue(i0 + 1, 1)
            process(i0, 0, (i0 + 2) % AGG_NCHUNK)
            process(i0 + 1, 1, (i0 + 3) % AGG_NCHUNK)
            wait_scatter(0)
            wait_src((i0 + 2) % AGG_NCHUNK, 0)
            issue((i0 + 2) % AGG_NCHUNK, 0)
            return 0
        lax.fori_loop(0, AGG_NCHUNK // 2, body, 0)

        wait_scatter(1)
        wait_src(1 % AGG_NCHUNK, 1)
        src_c, dst_c, wrow, rows, semg, sems, semw, semd, semi = slot(0)
        pltpu.make_async_copy(g_ref.at[src_c], rows, semg).wait()
        pltpu.make_async_copy(ewb_hbm.at[s, 0], wrow, semw).wait()
        pltpu.make_async_copy(dst_hbm.at[s, 0], dst_c, semd).wait()

    @pl.when(c == 0)
    def _():
        edge_loop(ga_hbm)

    @pl.when(c == 1)
    def _():
        edge_loop(gb_hbm)

    plsc.subcore_barrier()

    def wb(out_ref):
        stage = rows2.at[pl.ds(0, WB_CHUNK)]
        for i in range(SLAB // WB_CHUNK):
            base = s * SLAB + i * WB_CHUNK
            pltpu.sync_copy(acc.at[pl.ds(base, WB_CHUNK)], stage)
            pltpu.sync_copy(stage, out_ref.at[pl.ds(base, WB_CHUNK)])

    @pl.when(c == 0)
    def _():
        wb(outa_hbm)

    @pl.when(c == 1)
    def _():
        wb(outb_hbm)


@functools.partial(
    pl.kernel,
    out_type=(jax.ShapeDtypeStruct((NPAD, H2), jnp.float32),
              jax.ShapeDtypeStruct((NPAD, H2), jnp.float32)),
    mesh=_sc_mesh,
    scratch_types=[
        pltpu.VMEM((4, CA), jnp.int32),
        pltpu.VMEM((2 * CA, L), jnp.float32),
        pltpu.VMEM((2 * CA, H2), jnp.float32),
        pltpu.SemaphoreType.DMA,
        pltpu.SemaphoreType.DMA,
        pltpu.SemaphoreType.DMA,
        pltpu.SemaphoreType.DMA,
        pltpu.SemaphoreType.DMA,
        pltpu.SemaphoreType.DMA,
        pltpu.SemaphoreType.DMA,
        pltpu.SemaphoreType.DMA,
        pltpu.SemaphoreType.DMA,
        pltpu.SemaphoreType.DMA,
        pltpu.VMEM_SHARED((NPAD, H2), jnp.float32),
    ],
)
def _agg_call(ga_hbm, gb_hbm, src_hbm, dst_hbm, ewb_hbm, z2_hbm,
              outa_hbm, outb_hbm,
              idx4, wrow2, rows2,
              semg_a, semg_b, sems_a, sems_b, semw_a, semw_b, semd_a, semd_b,
              semi_a, semi_b,
              acc):
    _agg_body(ga_hbm, gb_hbm, src_hbm, dst_hbm, ewb_hbm, z2_hbm,
              outa_hbm, outb_hbm,
              idx4, wrow2, rows2,
              semg_a, semg_b, sems_a, sems_b, semw_a, semw_b, semd_a, semd_b,
              semi_a, semi_b,
              acc)


BR = 2000


def _dinv_body(degp_ref, dinv_ref):
    d = degp_ref[0, :] + degp_ref[1, :] + 1.0
    dinv_ref[...] = lax.rsqrt(d)[:, None]


def _dinv_call(deg_p):
    return pl.pallas_call(
        _dinv_body,
        out_shape=jax.ShapeDtypeStruct((NPAD, 1), jnp.float32),
    )(deg_p)


def _t1_body(x_ref, w_ref, dinv_ref, glo_ref, ghi_ref):
    h = jnp.dot(x_ref[...], w_ref[...], preferred_element_type=jnp.float32)
    g = h * dinv_ref[...]
    glo_ref[...] = g[:, :H2]
    ghi_ref[...] = g[:, H2:]


def _t1_call(x, W0, dinv):
    return pl.pallas_call(
        _t1_body,
        grid=(N // BR,),
        in_specs=[
            pl.BlockSpec((BR, IN_DIM), lambda i: (i, 0)),
            pl.BlockSpec((IN_DIM, H1), lambda i: (0, 0)),
            pl.BlockSpec((BR, 1), lambda i: (i, 0)),
        ],
        out_specs=(
            pl.BlockSpec((BR, H2), lambda i: (i, 0)),
            pl.BlockSpec((BR, H2), lambda i: (i, 0)),
        ),
        out_shape=(jax.ShapeDtypeStruct((N, H2), jnp.float32),
                   jax.ShapeDtypeStruct((N, H2), jnp.float32)),
    )(x, W0, dinv)


def _t2_body(alo_ref, ahi_ref, glo_ref, ghi_ref, dinv_ref, b0_ref, wc_ref,
             g1_ref, g2_ref):
    dinv = dinv_ref[...]
    hlo = (alo_ref[...] + glo_ref[...]) * dinv
    hhi = (ahi_ref[...] + ghi_ref[...]) * dinv
    h = jnp.concatenate([hlo, hhi], axis=1) + b0_ref[...]
    h = jnp.maximum(h, 0.0)
    m = jnp.dot(h, wc_ref[...], preferred_element_type=jnp.float32)
    g1_ref[...] = m[:, :H2] * dinv
    g2_ref[...] = m[:, H2:] * dinv


def _t2_call(alo, ahi, glo, ghi, dinv, b0, Wc):
    return pl.pallas_call(
        _t2_body,
        grid=(N // BR,),
        in_specs=[
            pl.BlockSpec((BR, H2), lambda i: (i, 0)),
            pl.BlockSpec((BR, H2), lambda i: (i, 0)),
            pl.BlockSpec((BR, H2), lambda i: (i, 0)),
            pl.BlockSpec((BR, H2), lambda i: (i, 0)),
            pl.BlockSpec((BR, 1), lambda i: (i, 0)),
            pl.BlockSpec((1, H1), lambda i: (0, 0)),
            pl.BlockSpec((H1, 2 * H2), lambda i: (0, 0)),
        ],
        out_specs=(
            pl.BlockSpec((BR, H2), lambda i: (i, 0)),
            pl.BlockSpec((BR, H2), lambda i: (i, 0)),
        ),
        out_shape=(jax.ShapeDtypeStruct((N, H2), jnp.float32),
                   jax.ShapeDtypeStruct((N, H2), jnp.float32)),
    )(alo, ahi, glo, ghi, dinv, b0, Wc)


def _t3_body(a1_ref, g1_ref, a2_ref, g2_ref, dinv_ref, b1_ref, b2_ref,
             noise_ref, z_ref, zb_ref):
    dinv = dinv_ref[...]
    mean = (a1_ref[...] + g1_ref[...]) * dinv + b1_ref[...]
    log_std = (a2_ref[...] + g2_ref[...]) * dinv + b2_ref[...]
    z = mean + noise_ref[...] * jnp.exp(log_std)
    z_ref[...] = z
    zb_ref[...] = z.astype(jnp.bfloat16)


def _t3_call(a1, g1, a2, g2, dinv, b1, b2, noise):
    return pl.pallas_call(
        _t3_body,
        grid=(N // BR,),
        in_specs=[
            pl.BlockSpec((BR, H2), lambda i: (i, 0)),
            pl.BlockSpec((BR, H2), lambda i: (i, 0)),
            pl.BlockSpec((BR, H2), lambda i: (i, 0)),
            pl.BlockSpec((BR, H2), lambda i: (i, 0)),
            pl.BlockSpec((BR, 1), lambda i: (i, 0)),
            pl.BlockSpec((1, H2), lambda i: (0, 0)),
            pl.BlockSpec((1, H2), lambda i: (0, 0)),
            pl.BlockSpec((BR, H2), lambda i: (i, 0)),
        ],
        out_specs=(pl.BlockSpec((BR, H2), lambda i: (i, 0)),
                   pl.BlockSpec((BR, H2), lambda i: (i, 0))),
        out_shape=(jax.ShapeDtypeStruct((N, H2), jnp.float32),
                   jax.ShapeDtypeStruct((N, H2), jnp.bfloat16)),
    )(a1, g1, a2, g2, dinv, b1, b2, noise)


DEC_BM = 400


def _decoder_body(zi_ref, zj_ref, out_ref):
    acc = jax.lax.dot_general(
        zi_ref[...], zj_ref[...],
        (((1,), (1,)), ((), ())),
        preferred_element_type=jnp.float32,
    )
    out_ref[...] = jax.nn.sigmoid(acc)


def _decoder(zb):
    n = zb.shape[0]
    return pl.pallas_call(
        _decoder_body,
        grid=(n // DEC_BM,),
        in_specs=[
            pl.BlockSpec((DEC_BM, H2), lambda i: (i, 0)),
            pl.BlockSpec((n, H2), lambda i: (0, 0)),
        ],
        out_specs=pl.BlockSpec((DEC_BM, n), lambda i: (i, 0)),
        out_shape=jax.ShapeDtypeStruct((n, n), jnp.float32),
    )(zb, zb)


def kernel(x, edge_index, edge_weight, W0, b0, W1, b1, W2, b2):
    src = edge_index[0]
    dst = edge_index[1]
    n = x.shape[0]

    pad = EPAD - E
    pad_idx = (jnp.arange(pad, dtype=jnp.int32) * 37) % N
    src_p = jnp.concatenate([src, pad_idx])
    dst_p = jnp.concatenate([dst, pad_idx])
    ew_p = jnp.concatenate([edge_weight, jnp.zeros((pad,), jnp.float32)])

    dst_deg = dst_p.reshape(NC * NS, DEG_NCHUNK, ECHUNK)
    ew_deg = ew_p.reshape(NC * NS, DEG_NCHUNK, ECHUNK)
    src_agg = src_p.reshape(NS, AGG_NCHUNK, CA)
    dst_agg = dst_p.reshape(NS, AGG_NCHUNK, CA)
    ew_bc = jnp.broadcast_to(
        ew_p[:, None], (EPAD, L)).reshape(NS, AGG_NCHUNK, CA, L)

    z1 = jnp.zeros((SLAB,), jnp.float32)
    z2 = jnp.zeros((SLAB, H2), jnp.float32)

    deg0, deg1 = _deg_call(dst_deg, ew_deg, z1)
    dinv_pad = _dinv_call(jnp.stack([deg0, deg1]))
    dinv = dinv_pad[:N]

    g0_lo, g0_hi = _t1_call(x, W0, dinv)
    a0_lo, a0_hi = _agg_call(g0_lo, g0_hi, src_agg, dst_agg, ew_bc, z2)
    a0_lo, a0_hi = a0_lo[:N], a0_hi[:N]

    Wc = jnp.concatenate([W1, W2], axis=1)
    g1, g2 = _t2_call(a0_lo, a0_hi, g0_lo, g0_hi, dinv,
                      b0.reshape(1, H1), Wc)
    a1, a2 = _agg_call(g1, g2, src_agg, dst_agg, ew_bc, z2)
    a1, a2 = a1[:N], a2[:N]

    noise = jax.random.normal(jax.random.key(42), (n, H2), dtype=x.dtype)
    z, z_bf16 = _t3_call(a1, g1, a2, g2, dinv,
                         b1.reshape(1, H2), b2.reshape(1, H2), noise)

    adj_rec = _decoder(z_bf16)
    return (adj_rec, z)
